# baseline (speedup 1.0000x reference)
import functools

import jax
import jax.numpy as jnp
from jax import lax
from jax.experimental import pallas as pl
from jax.experimental.pallas import tpu as pltpu
from jax.experimental.pallas import tpu_sc as plsc

N = 10000
E = 320000
H = 128


_BN = 2000
_BE = 3200


def _dot(a, b):
    return jnp.dot(a, b, preferred_element_type=jnp.float32)


def _ln(h2, g, beta):
    ones = jnp.ones((H, 1), jnp.bfloat16)
    s1 = jnp.dot(h2.astype(jnp.bfloat16), ones,
                 preferred_element_type=jnp.float32)
    s2 = jnp.dot((h2 * h2).astype(jnp.bfloat16), ones,
                 preferred_element_type=jnp.float32)
    mu = s1 * (1.0 / H)
    var = s2 * (1.0 / H) - mu * mu
    return (h2 - mu) * lax.rsqrt(var + 1e-5) * g + beta


def _mlp_ln_tail(h1_pre, w2, b2, g, beta):
    h = jnp.maximum(h1_pre, 0.0)
    return _ln(_dot(h, w2) + b2, g, beta)


def _enc_body(x_ref, w1, b1, w2, b2, g, beta, o_ref):
    o_ref[...] = _mlp_ln_tail(_dot(x_ref[...], w1[...]) + b1[...],
                              w2[...], b2[...], g[...], beta[...])


def _pre_body(xn_ref, w1a, w1b, b1, a_ref, b_ref):
    xn = xn_ref[...]
    a_ref[...] = _dot(xn, w1a[...]) + b1[...]
    b_ref[...] = _dot(xn, w1b[...])


def _dot16(a, b):
    return jnp.dot(a.astype(jnp.bfloat16), b.astype(jnp.bfloat16),
                   preferred_element_type=jnp.float32)


def _edge_body(gi_ref, gj_ref, xe_ref, w1c, w2, b2, g, beta, o_ref):
    xe = xe_ref[...]
    h1 = gi_ref[...] + gj_ref[...] + _dot16(xe, w1c[...])
    h = jnp.maximum(h1, 0.0)
    o_ref[...] = _ln(_dot16(h, w2[...]) + b2[...], g[...], beta[...]) + xe


def _node_body(p0, p1, xn_ref, wna, wnb, b1, w2, b2, g, beta, o_ref):
    xn = xn_ref[...]
    msg = (p0[0] + p0[1]) + (p1[0] + p1[1])
    h1 = _dot(msg, wna[...]) + _dot(xn, wnb[...]) + b1[...]
    o_ref[...] = _mlp_ln_tail(h1, w2[...], b2[...], g[...], beta[...]) + xn


def _dec_body(xn_ref, w1, b1, w2, b2, o_ref):
    h = jnp.maximum(_dot(xn_ref[...], w1[...]) + b1[...], 0.0)
    o_ref[...] = _dot(h, w2[...]) + b2[...]


def _full(shape):
    return pl.BlockSpec(shape, lambda i: (0,) * len(shape))


def _rows(block, ncols):
    return pl.BlockSpec((block, ncols), lambda i: (i, 0))


def _tc_call(body, grid, in_specs, out_specs, out_shape):
    return pl.pallas_call(body, grid=(grid,), in_specs=in_specs,
                          out_specs=out_specs, out_shape=out_shape)


_CH = 128
_NCH = E // _CH
_NW = 32
_PER_W = -(-_NCH // _NW)
_RPS = 624
_REM = N - 16 * _RPS
_ZR = 104

@functools.cache
def _mesh():
    return plsc.VectorSubcoreMesh(core_axis_name="c", subcore_axis_name="s")


_MC = 128
_NSUB = _MC // _CH
_NSLOT = 3


def _gather_pipe(tab_hbm, ix_hbm, out_hbm, idx, rows, sem_idx, sem_g, sem_st,
                 s):
    nch = ix_hbm.shape[0] // _MC
    per_w = -(-nch // 16)

    def chunk(m):
        return m * 16 + s

    def valid(m):
        return chunk(m) < nch

    def issue_idx(m, k):
        pltpu.async_copy(ix_hbm.at[pl.ds(chunk(m) * _MC, _MC)], idx.at[k],
                         sem_idx)

    for m0 in range(min(2, per_w)):
        @pl.when(valid(m0))
        def _(m0=m0):
            issue_idx(m0, m0)

    @pl.loop(0, per_w + 1)
    def _(m):
        k = m % _NSLOT
        kp = (m + _NSLOT - 1) % _NSLOT

        @pl.when((m < per_w) & valid(m))
        def _():
            @pl.when(m >= _NSLOT)
            def _():
                pltpu.make_async_copy(rows.at[k], out_hbm.at[pl.ds(0, _MC)],
                                      sem_st).wait()
            pltpu.make_async_copy(ix_hbm.at[pl.ds(0, _MC)], idx.at[k],
                                  sem_idx).wait()
            for u in range(_NSUB):
                pltpu.async_copy(tab_hbm.at[idx.at[k, pl.ds(u * _CH, _CH)]],
                                 rows.at[k, pl.ds(u * _CH, _CH)], sem_g)

        @pl.when((m >= 1) & valid(m - 1))
        def _():
            for u in range(_NSUB):
                pltpu.make_async_copy(tab_hbm.at[idx.at[kp, pl.ds(0, _CH)]],
                                      rows.at[kp, pl.ds(0, _CH)],
                                      sem_g).wait()
            pltpu.async_copy(rows.at[kp],
                             out_hbm.at[pl.ds(chunk(m - 1) * _MC, _MC)],
                             sem_st)

        @pl.when((m + 2 < per_w) & valid(m + 2))
        def _():
            issue_idx(m + 2, (m + 2) % _NSLOT)

    for q in range(max(0, per_w - _NSLOT - 1), per_w):
        undrained = valid(q)
        if q + _NSLOT < per_w:
            undrained = undrained & jnp.logical_not(valid(q + _NSLOT))

        @pl.when(undrained)
        def _(q=q):
            pltpu.make_async_copy(rows.at[q % _NSLOT],
                                  out_hbm.at[pl.ds(0, _MC)], sem_st).wait()


def _gather_body(a_hbm, b_hbm, ti_hbm, si_hbm, gi_hbm, gj_hbm,
                 idx, rows, sem_idx, sem_g, sem_st):
    c = lax.axis_index("c")
    s = lax.axis_index("s")

    @pl.when(c == 0)
    def _():
        _gather_pipe(a_hbm, ti_hbm, gi_hbm, idx, rows, sem_idx, sem_g,
                     sem_st, s)

    @pl.when(c == 1)
    def _():
        _gather_pipe(b_hbm, si_hbm, gj_hbm, idx, rows, sem_idx, sem_g,
                     sem_st, s)


@jax.jit
def _sc_gather(a, b, tgt, src):
    ne = tgt.shape[0]
    k = pl.kernel(
        _gather_body,
        out_type=(jax.ShapeDtypeStruct((ne, H), jnp.float32),
                  jax.ShapeDtypeStruct((ne, H), jnp.float32)),
        mesh=_mesh(),
        scratch_types=[
            pltpu.VMEM((_NSLOT, _MC), jnp.int32),
            pltpu.VMEM((_NSLOT, _MC, H), jnp.float32),
            pltpu.SemaphoreType.DMA,
            pltpu.SemaphoreType.DMA,
            pltpu.SemaphoreType.DMA,
        ],
    )
    return k(a, b, tgt, src)


def _scatter_body(ye_hbm, si_hbm, out_hbm, idx, rows, zbuf, acc, sem_ld,
                  sem_sc):
    ne = si_hbm.shape[0]
    nch = ne // _CH
    per_w = -(-nch // _NW)
    c = lax.axis_index("c")
    s = lax.axis_index("s")
    w = s * 2 + c

    @pl.loop(0, _ZR)
    def _(r):
        @pl.loop(0, H, step=16)
        def _(l):
            zbuf[pl.ds(r, 1), pl.ds(l, 16)] = jnp.zeros((1, 16), jnp.float32)

    @pl.loop(0, _RPS, step=_ZR)
    def _(r):
        pltpu.sync_copy(zbuf, acc.at[pl.ds(s * _RPS + r, _ZR)])

    @pl.when(s == 0)
    def _():
        pltpu.sync_copy(zbuf.at[pl.ds(0, _REM)], acc.at[pl.ds(16 * _RPS, _REM)])

    plsc.subcore_barrier()

    def chunk(m):
        return m * _NW + w

    def valid(m):
        return chunk(m) < nch

    def issue_load(m, k):
        base = chunk(m) * _CH
        pltpu.async_copy(si_hbm.at[pl.ds(base, _CH)], idx.at[k], sem_ld)
        pltpu.async_copy(ye_hbm.at[pl.ds(base, _CH)], rows.at[k], sem_ld)

    @pl.when(valid(0))
    def _():
        issue_load(0, 0)

    @pl.loop(0, per_w + 1)
    def _(m):
        k = m % 2
        kp = (m + 1) % 2

        @pl.when((m < per_w) & valid(m))
        def _():
            pltpu.make_async_copy(si_hbm.at[pl.ds(0, _CH)], idx.at[k],
                                  sem_ld).wait()
            pltpu.make_async_copy(ye_hbm.at[pl.ds(0, _CH)], rows.at[k],
                                  sem_ld).wait()
            pltpu.async_copy(rows.at[k], acc.at[idx.at[k]], sem_sc, add=True)

        @pl.when((m >= 1) & valid(m - 1))
        def _():
            pltpu.make_async_copy(rows.at[kp], acc.at[pl.ds(0, _CH)],
                                  sem_sc).wait()

        @pl.when((m + 1 < per_w) & valid(m + 1))
        def _():
            issue_load(m + 1, kp)

    plsc.subcore_barrier()
    pltpu.sync_copy(acc.at[pl.ds(s * _RPS, _RPS)],
                    out_hbm.at[c, pl.ds(s * _RPS, _RPS)])

    @pl.when(s == 0)
    def _():
        pltpu.sync_copy(acc.at[pl.ds(16 * _RPS, _REM)],
                        out_hbm.at[c, pl.ds(16 * _RPS, _REM)])


@jax.jit
def _sc_scatter(ye, src):
    k = pl.kernel(
        _scatter_body,
        out_type=jax.ShapeDtypeStruct((2, N, H), jnp.float32),
        mesh=_mesh(),
        scratch_types=[
            pltpu.VMEM((2, _CH), jnp.int32),
            pltpu.VMEM((2, _CH, H), jnp.float32),
            pltpu.VMEM((_ZR, H), jnp.float32),
            pltpu.VMEM_SHARED((N, H), jnp.float32),
            pltpu.SemaphoreType.DMA,
            pltpu.SemaphoreType.DMA,
        ],
    )
    return k(ye, src)


def _b(v):
    return v.reshape(1, H)


_NP = 2
_EH = E // _NP


def kernel(x, edge_attr, edge_index, params):
    f32 = jnp.float32
    ngrid = N // _BN
    eg_h = _EH // _BE
    src = [lax.slice_in_dim(edge_index[0], h * _EH, (h + 1) * _EH)
           for h in range(_NP)]
    tgt = [lax.slice_in_dim(edge_index[1], h * _EH, (h + 1) * _EH)
           for h in range(_NP)]

    def enc(inp, p, block, grid, fin, nrows, blk_off=0):
        return _tc_call(
            _enc_body, grid,
            [pl.BlockSpec((block, fin), lambda i: (i + blk_off, 0)),
             _full((fin, H)), _full((1, H)),
             _full((H, H)), _full((1, H)), _full((1, H)), _full((1, H))],
            _rows(block, H), jax.ShapeDtypeStruct((nrows, H), f32),
        )(inp, p["l1"]["w"], _b(p["l1"]["b"]), p["l2"]["w"], _b(p["l2"]["b"]),
          _b(p["g"]), _b(p["beta"]))

    xn = enc(x, params["node_enc"], _BN, ngrid, x.shape[1], N)
    ci_e = edge_attr.shape[1]
    xe = [enc(edge_attr, params["edge_enc"], _BE, eg_h, ci_e, _EH,
              blk_off=h * eg_h) for h in range(_NP)]

    for lp in params["layers"]:
        ew = lp["edge_mlp"]
        w1 = ew["l1"]["w"]
        a_tab, b_tab = _tc_call(
            _pre_body, ngrid,
            [_rows(_BN, H), _full((H, H)), _full((H, H)), _full((1, H))],
            (_rows(_BN, H), _rows(_BN, H)),
            (jax.ShapeDtypeStruct((N, H), f32),
             jax.ShapeDtypeStruct((N, H), f32)),
        )(xn, w1[:H], w1[H:2 * H], _b(ew["l1"]["b"]))

        g_h = [_sc_gather(a_tab, b_tab, tgt[h], src[h]) for h in range(_NP)]

        ye = [_tc_call(
            _edge_body, eg_h,
            [_rows(_BE, H)] * 3 + [_full((H, H)), _full((H, H)),
                                   _full((1, H)), _full((1, H)), _full((1, H))],
            _rows(_BE, H), jax.ShapeDtypeStruct((_EH, H), f32),
        )(g_h[h][0], g_h[h][1], xe[h], w1[2 * H:], ew["l2"]["w"],
          _b(ew["l2"]["b"]), _b(ew["g"]), _b(ew["beta"])) for h in range(_NP)]

        p_sum = [_sc_scatter(ye[h], src[h]) for h in range(_NP)]

        nw = lp["node_mlp"]
        wn1 = nw["l1"]["w"]
        p_spec = pl.BlockSpec((2, _BN, H), lambda i: (0, i, 0))
        xn = _tc_call(
            _node_body, ngrid,
            [p_spec] * _NP + [_rows(_BN, H),
             _full((H, H)), _full((H, H)),
             _full((1, H)), _full((H, H)), _full((1, H)), _full((1, H)),
             _full((1, H))],
            _rows(_BN, H), jax.ShapeDtypeStruct((N, H), f32),
        )(p_sum[0], p_sum[1], xn, wn1[:H], wn1[H:],
          _b(nw["l1"]["b"]), nw["l2"]["w"], _b(nw["l2"]["b"]), _b(nw["g"]),
          _b(nw["beta"]))
        xe = ye

    dec = params["dec"]
    out = _tc_call(
        _dec_body, ngrid,
        [_rows(_BN, H), _full((H, H)), _full((1, H)), _full((H, H)),
         _full((1, H))],
        _rows(_BN, H), jax.ShapeDtypeStruct((N, dec["l2"]["w"].shape[1]), f32),
    )(xn, dec["l1"]["w"], _b(dec["l1"]["b"]), dec["l2"]["w"],
      _b(dec["l2"]["b"]))
    return out


# speedup vs baseline: 1.0401x; 1.0401x over previous
import functools

import jax
import jax.numpy as jnp
from jax import lax
from jax.experimental import pallas as pl
from jax.experimental.pallas import tpu as pltpu
from jax.experimental.pallas import tpu_sc as plsc

N = 10000
E = 320000
H = 128


_BN = 2000
_BE = 8000


def _dot(a, b):
    return jnp.dot(a, b, preferred_element_type=jnp.float32)


def _ln(h2, g, beta):
    mu = jnp.mean(h2, axis=-1, keepdims=True)
    var = jnp.mean((h2 - mu) ** 2, axis=-1, keepdims=True)
    return (h2 - mu) * lax.rsqrt(var + 1e-5) * g + beta


def _mlp_ln_tail(h1_pre, w2, b2, g, beta):
    h = jnp.maximum(h1_pre, 0.0)
    return _ln(_dot(h, w2) + b2, g, beta)


def _enc_body(x_ref, w1, b1, w2, b2, g, beta, o_ref):
    o_ref[...] = _mlp_ln_tail(_dot(x_ref[...], w1[...]) + b1[...],
                              w2[...], b2[...], g[...], beta[...])


def _pre_body(xn_ref, w1a, w1b, b1, a_ref, b_ref):
    xn = xn_ref[...]
    a_ref[...] = _dot(xn, w1a[...]) + b1[...]
    b_ref[...] = _dot(xn, w1b[...])


def _dot16(a, b):
    return jnp.dot(a.astype(jnp.bfloat16), b.astype(jnp.bfloat16),
                   preferred_element_type=jnp.float32)


def _edge_body(gi_ref, gj_ref, xe_ref, w1c, w2, b2, g, beta, o_ref):
    xe = xe_ref[...]
    h1 = gi_ref[...] + gj_ref[...] + _dot16(xe, w1c[...])
    h = jnp.maximum(h1, 0.0)
    o_ref[...] = _ln(_dot16(h, w2[...]) + b2[...], g[...], beta[...]) + xe


def _node_body(p0, p1, xn_ref, wna, wnb, b1, w2, b2, g, beta, o_ref):
    xn = xn_ref[...]
    msg = (p0[0] + p0[1]) + (p1[0] + p1[1])
    h1 = _dot(msg, wna[...]) + _dot(xn, wnb[...]) + b1[...]
    o_ref[...] = _mlp_ln_tail(h1, w2[...], b2[...], g[...], beta[...]) + xn


def _dec_body(xn_ref, w1, b1, w2, b2, o_ref):
    h = jnp.maximum(_dot(xn_ref[...], w1[...]) + b1[...], 0.0)
    o_ref[...] = _dot(h, w2[...]) + b2[...]


def _full(shape):
    return pl.BlockSpec(shape, lambda i: (0,) * len(shape))


def _rows(block, ncols):
    return pl.BlockSpec((block, ncols), lambda i: (i, 0))


def _tc_call(body, grid, in_specs, out_specs, out_shape):
    return pl.pallas_call(body, grid=(grid,), in_specs=in_specs,
                          out_specs=out_specs, out_shape=out_shape)


_CH = 128
_NCH = E // _CH
_NW = 32
_PER_W = -(-_NCH // _NW)
_RPS = 624
_REM = N - 16 * _RPS
_ZR = 104

@functools.cache
def _mesh():
    return plsc.VectorSubcoreMesh(core_axis_name="c", subcore_axis_name="s")


_MC = 128
_NSUB = _MC // _CH
_NSLOT = 3


def _gather_pipe(tab_hbm, ix_hbm, out_hbm, idx, rows, sem_idx, sem_g, sem_st,
                 s):
    nch = ix_hbm.shape[0] // _MC
    per_w = -(-nch // 16)

    def chunk(m):
        return m * 16 + s

    def valid(m):
        return chunk(m) < nch

    def issue_idx(m, k):
        pltpu.async_copy(ix_hbm.at[pl.ds(chunk(m) * _MC, _MC)], idx.at[k],
                         sem_idx)

    for m0 in range(min(2, per_w)):
        @pl.when(valid(m0))
        def _(m0=m0):
            issue_idx(m0, m0)

    @pl.loop(0, per_w + 1)
    def _(m):
        k = m % _NSLOT
        kp = (m + _NSLOT - 1) % _NSLOT

        @pl.when((m < per_w) & valid(m))
        def _():
            @pl.when(m >= _NSLOT)
            def _():
                pltpu.make_async_copy(rows.at[k], out_hbm.at[pl.ds(0, _MC)],
                                      sem_st).wait()
            pltpu.make_async_copy(ix_hbm.at[pl.ds(0, _MC)], idx.at[k],
                                  sem_idx).wait()
            for u in range(_NSUB):
                pltpu.async_copy(tab_hbm.at[idx.at[k, pl.ds(u * _CH, _CH)]],
                                 rows.at[k, pl.ds(u * _CH, _CH)], sem_g)

        @pl.when((m >= 1) & valid(m - 1))
        def _():
            for u in range(_NSUB):
                pltpu.make_async_copy(tab_hbm.at[idx.at[kp, pl.ds(0, _CH)]],
                                      rows.at[kp, pl.ds(0, _CH)],
                                      sem_g).wait()
            pltpu.async_copy(rows.at[kp],
                             out_hbm.at[pl.ds(chunk(m - 1) * _MC, _MC)],
                             sem_st)

        @pl.when((m + 2 < per_w) & valid(m + 2))
        def _():
            issue_idx(m + 2, (m + 2) % _NSLOT)

    for q in range(max(0, per_w - _NSLOT - 1), per_w):
        undrained = valid(q)
        if q + _NSLOT < per_w:
            undrained = undrained & jnp.logical_not(valid(q + _NSLOT))

        @pl.when(undrained)
        def _(q=q):
            pltpu.make_async_copy(rows.at[q % _NSLOT],
                                  out_hbm.at[pl.ds(0, _MC)], sem_st).wait()


def _gather_body(a_hbm, b_hbm, ti_hbm, si_hbm, gi_hbm, gj_hbm,
                 idx, rows, sem_idx, sem_g, sem_st):
    c = lax.axis_index("c")
    s = lax.axis_index("s")

    @pl.when(c == 0)
    def _():
        _gather_pipe(a_hbm, ti_hbm, gi_hbm, idx, rows, sem_idx, sem_g,
                     sem_st, s)

    @pl.when(c == 1)
    def _():
        _gather_pipe(b_hbm, si_hbm, gj_hbm, idx, rows, sem_idx, sem_g,
                     sem_st, s)


@jax.jit
def _sc_gather(a, b, tgt, src):
    ne = tgt.shape[0]
    k = pl.kernel(
        _gather_body,
        out_type=(jax.ShapeDtypeStruct((ne, H), jnp.float32),
                  jax.ShapeDtypeStruct((ne, H), jnp.float32)),
        mesh=_mesh(),
        scratch_types=[
            pltpu.VMEM((_NSLOT, _MC), jnp.int32),
            pltpu.VMEM((_NSLOT, _MC, H), jnp.float32),
            pltpu.SemaphoreType.DMA,
            pltpu.SemaphoreType.DMA,
            pltpu.SemaphoreType.DMA,
        ],
    )
    return k(a, b, tgt, src)


def _scatter_body(ye_hbm, si_hbm, out_hbm, idx, rows, zbuf, acc, sem_ld,
                  sem_sc):
    ne = si_hbm.shape[0]
    nch = ne // _CH
    per_w = -(-nch // _NW)
    c = lax.axis_index("c")
    s = lax.axis_index("s")
    w = s * 2 + c

    @pl.loop(0, _ZR)
    def _(r):
        @pl.loop(0, H, step=16)
        def _(l):
            zbuf[pl.ds(r, 1), pl.ds(l, 16)] = jnp.zeros((1, 16), jnp.float32)

    @pl.loop(0, _RPS, step=_ZR)
    def _(r):
        pltpu.sync_copy(zbuf, acc.at[pl.ds(s * _RPS + r, _ZR)])

    @pl.when(s == 0)
    def _():
        pltpu.sync_copy(zbuf.at[pl.ds(0, _REM)], acc.at[pl.ds(16 * _RPS, _REM)])

    plsc.subcore_barrier()

    def chunk(m):
        return m * _NW + w

    def valid(m):
        return chunk(m) < nch

    def issue_load(m, k):
        base = chunk(m) * _CH
        pltpu.async_copy(si_hbm.at[pl.ds(base, _CH)], idx.at[k], sem_ld)
        pltpu.async_copy(ye_hbm.at[pl.ds(base, _CH)], rows.at[k], sem_ld)

    @pl.when(valid(0))
    def _():
        issue_load(0, 0)

    @pl.loop(0, per_w + 1)
    def _(m):
        k = m % 2
        kp = (m + 1) % 2

        @pl.when((m < per_w) & valid(m))
        def _():
            pltpu.make_async_copy(si_hbm.at[pl.ds(0, _CH)], idx.at[k],
                                  sem_ld).wait()
            pltpu.make_async_copy(ye_hbm.at[pl.ds(0, _CH)], rows.at[k],
                                  sem_ld).wait()
            pltpu.async_copy(rows.at[k], acc.at[idx.at[k]], sem_sc, add=True)

        @pl.when((m >= 1) & valid(m - 1))
        def _():
            pltpu.make_async_copy(rows.at[kp], acc.at[pl.ds(0, _CH)],
                                  sem_sc).wait()

        @pl.when((m + 1 < per_w) & valid(m + 1))
        def _():
            issue_load(m + 1, kp)

    plsc.subcore_barrier()
    pltpu.sync_copy(acc.at[pl.ds(s * _RPS, _RPS)],
                    out_hbm.at[c, pl.ds(s * _RPS, _RPS)])

    @pl.when(s == 0)
    def _():
        pltpu.sync_copy(acc.at[pl.ds(16 * _RPS, _REM)],
                        out_hbm.at[c, pl.ds(16 * _RPS, _REM)])


@jax.jit
def _sc_scatter(ye, src):
    k = pl.kernel(
        _scatter_body,
        out_type=jax.ShapeDtypeStruct((2, N, H), jnp.float32),
        mesh=_mesh(),
        scratch_types=[
            pltpu.VMEM((2, _CH), jnp.int32),
            pltpu.VMEM((2, _CH, H), jnp.float32),
            pltpu.VMEM((_ZR, H), jnp.float32),
            pltpu.VMEM_SHARED((N, H), jnp.float32),
            pltpu.SemaphoreType.DMA,
            pltpu.SemaphoreType.DMA,
        ],
    )
    return k(ye, src)


def _b(v):
    return v.reshape(1, H)


_NP = 2
_EH = E // _NP


def kernel(x, edge_attr, edge_index, params):
    f32 = jnp.float32
    ngrid = N // _BN
    eg_h = _EH // _BE
    src = [lax.slice_in_dim(edge_index[0], h * _EH, (h + 1) * _EH)
           for h in range(_NP)]
    tgt = [lax.slice_in_dim(edge_index[1], h * _EH, (h + 1) * _EH)
           for h in range(_NP)]

    def enc(inp, p, block, grid, fin, nrows, blk_off=0):
        return _tc_call(
            _enc_body, grid,
            [pl.BlockSpec((block, fin), lambda i: (i + blk_off, 0)),
             _full((fin, H)), _full((1, H)),
             _full((H, H)), _full((1, H)), _full((1, H)), _full((1, H))],
            _rows(block, H), jax.ShapeDtypeStruct((nrows, H), f32),
        )(inp, p["l1"]["w"], _b(p["l1"]["b"]), p["l2"]["w"], _b(p["l2"]["b"]),
          _b(p["g"]), _b(p["beta"]))

    xn = enc(x, params["node_enc"], _BN, ngrid, x.shape[1], N)
    ci_e = edge_attr.shape[1]
    xe = [enc(edge_attr, params["edge_enc"], _BE, eg_h, ci_e, _EH,
              blk_off=h * eg_h) for h in range(_NP)]

    for lp in params["layers"]:
        ew = lp["edge_mlp"]
        w1 = ew["l1"]["w"]
        a_tab, b_tab = _tc_call(
            _pre_body, ngrid,
            [_rows(_BN, H), _full((H, H)), _full((H, H)), _full((1, H))],
            (_rows(_BN, H), _rows(_BN, H)),
            (jax.ShapeDtypeStruct((N, H), f32),
             jax.ShapeDtypeStruct((N, H), f32)),
        )(xn, w1[:H], w1[H:2 * H], _b(ew["l1"]["b"]))

        g_h = [_sc_gather(a_tab, b_tab, tgt[h], src[h]) for h in range(_NP)]

        ye = [_tc_call(
            _edge_body, eg_h,
            [_rows(_BE, H)] * 3 + [_full((H, H)), _full((H, H)),
                                   _full((1, H)), _full((1, H)), _full((1, H))],
            _rows(_BE, H), jax.ShapeDtypeStruct((_EH, H), f32),
        )(g_h[h][0], g_h[h][1], xe[h], w1[2 * H:], ew["l2"]["w"],
          _b(ew["l2"]["b"]), _b(ew["g"]), _b(ew["beta"])) for h in range(_NP)]

        p_sum = [_sc_scatter(ye[h], src[h]) for h in range(_NP)]

        nw = lp["node_mlp"]
        wn1 = nw["l1"]["w"]
        p_spec = pl.BlockSpec((2, _BN, H), lambda i: (0, i, 0))
        xn = _tc_call(
            _node_body, ngrid,
            [p_spec] * _NP + [_rows(_BN, H),
             _full((H, H)), _full((H, H)),
             _full((1, H)), _full((H, H)), _full((1, H)), _full((1, H)),
             _full((1, H))],
            _rows(_BN, H), jax.ShapeDtypeStruct((N, H), f32),
        )(p_sum[0], p_sum[1], xn, wn1[:H], wn1[H:],
          _b(nw["l1"]["b"]), nw["l2"]["w"], _b(nw["l2"]["b"]), _b(nw["g"]),
          _b(nw["beta"]))
        xe = ye

    dec = params["dec"]
    out = _tc_call(
        _dec_body, ngrid,
        [_rows(_BN, H), _full((H, H)), _full((1, H)), _full((H, H)),
         _full((1, H))],
        _rows(_BN, H), jax.ShapeDtypeStruct((N, dec["l2"]["w"].shape[1]), f32),
    )(xn, dec["l1"]["w"], _b(dec["l1"]["b"]), dec["l2"]["w"],
      _b(dec["l2"]["b"]))
    return out


# speedup vs baseline: 1.0432x; 1.0030x over previous
import functools

import jax
import jax.numpy as jnp
from jax import lax
from jax.experimental import pallas as pl
from jax.experimental.pallas import tpu as pltpu
from jax.experimental.pallas import tpu_sc as plsc

N = 10000
E = 320000
H = 128


_BN = 2000
_BE = 10000


def _dot(a, b):
    return jnp.dot(a, b, preferred_element_type=jnp.float32)


def _ln(h2, g, beta):
    mu = jnp.mean(h2, axis=-1, keepdims=True)
    var = jnp.mean((h2 - mu) ** 2, axis=-1, keepdims=True)
    return (h2 - mu) * lax.rsqrt(var + 1e-5) * g + beta


def _mlp_ln_tail(h1_pre, w2, b2, g, beta):
    h = jnp.maximum(h1_pre, 0.0)
    return _ln(_dot(h, w2) + b2, g, beta)


def _enc_body(x_ref, w1, b1, w2, b2, g, beta, o_ref):
    o_ref[...] = _mlp_ln_tail(_dot(x_ref[...], w1[...]) + b1[...],
                              w2[...], b2[...], g[...], beta[...])


def _pre_body(xn_ref, w1a, w1b, b1, a_ref, b_ref):
    xn = xn_ref[...]
    a_ref[...] = _dot(xn, w1a[...]) + b1[...]
    b_ref[...] = _dot(xn, w1b[...])


def _dot16(a, b):
    return jnp.dot(a.astype(jnp.bfloat16), b.astype(jnp.bfloat16),
                   preferred_element_type=jnp.float32)


def _edge_body(gi_ref, gj_ref, xe_ref, w1c, w2, b2, g, beta, o_ref):
    xe = xe_ref[...]
    h1 = gi_ref[...] + gj_ref[...] + _dot16(xe, w1c[...])
    h = jnp.maximum(h1, 0.0)
    o_ref[...] = _ln(_dot16(h, w2[...]) + b2[...], g[...], beta[...]) + xe


def _node_body(p0, p1, xn_ref, wna, wnb, b1, w2, b2, g, beta, o_ref):
    xn = xn_ref[...]
    msg = (p0[0] + p0[1]) + (p1[0] + p1[1])
    h1 = _dot(msg, wna[...]) + _dot(xn, wnb[...]) + b1[...]
    o_ref[...] = _mlp_ln_tail(h1, w2[...], b2[...], g[...], beta[...]) + xn


def _dec_body(xn_ref, w1, b1, w2, b2, o_ref):
    h = jnp.maximum(_dot(xn_ref[...], w1[...]) + b1[...], 0.0)
    o_ref[...] = _dot(h, w2[...]) + b2[...]


def _full(shape):
    return pl.BlockSpec(shape, lambda i: (0,) * len(shape))


def _rows(block, ncols):
    return pl.BlockSpec((block, ncols), lambda i: (i, 0))


def _tc_call(body, grid, in_specs, out_specs, out_shape):
    return pl.pallas_call(body, grid=(grid,), in_specs=in_specs,
                          out_specs=out_specs, out_shape=out_shape)


_CH = 128
_NCH = E // _CH
_NW = 32
_PER_W = -(-_NCH // _NW)
_RPS = 624
_REM = N - 16 * _RPS
_ZR = 104

@functools.cache
def _mesh():
    return plsc.VectorSubcoreMesh(core_axis_name="c", subcore_axis_name="s")


_MC = 128
_NSUB = _MC // _CH
_NSLOT = 3


def _gather_pipe(tab_hbm, ix_hbm, out_hbm, idx, rows, sem_idx, sem_g, sem_st,
                 s):
    nch = ix_hbm.shape[0] // _MC
    per_w = -(-nch // 16)

    def chunk(m):
        return m * 16 + s

    def valid(m):
        return chunk(m) < nch

    def issue_idx(m, k):
        pltpu.async_copy(ix_hbm.at[pl.ds(chunk(m) * _MC, _MC)], idx.at[k],
                         sem_idx)

    for m0 in range(min(2, per_w)):
        @pl.when(valid(m0))
        def _(m0=m0):
            issue_idx(m0, m0)

    @pl.loop(0, per_w + 1)
    def _(m):
        k = m % _NSLOT
        kp = (m + _NSLOT - 1) % _NSLOT

        @pl.when((m < per_w) & valid(m))
        def _():
            @pl.when(m >= _NSLOT)
            def _():
                pltpu.make_async_copy(rows.at[k], out_hbm.at[pl.ds(0, _MC)],
                                      sem_st).wait()
            pltpu.make_async_copy(ix_hbm.at[pl.ds(0, _MC)], idx.at[k],
                                  sem_idx).wait()
            for u in range(_NSUB):
                pltpu.async_copy(tab_hbm.at[idx.at[k, pl.ds(u * _CH, _CH)]],
                                 rows.at[k, pl.ds(u * _CH, _CH)], sem_g)

        @pl.when((m >= 1) & valid(m - 1))
        def _():
            for u in range(_NSUB):
                pltpu.make_async_copy(tab_hbm.at[idx.at[kp, pl.ds(0, _CH)]],
                                      rows.at[kp, pl.ds(0, _CH)],
                                      sem_g).wait()
            pltpu.async_copy(rows.at[kp],
                             out_hbm.at[pl.ds(chunk(m - 1) * _MC, _MC)],
                             sem_st)

        @pl.when((m + 2 < per_w) & valid(m + 2))
        def _():
            issue_idx(m + 2, (m + 2) % _NSLOT)

    for q in range(max(0, per_w - _NSLOT - 1), per_w):
        undrained = valid(q)
        if q + _NSLOT < per_w:
            undrained = undrained & jnp.logical_not(valid(q + _NSLOT))

        @pl.when(undrained)
        def _(q=q):
            pltpu.make_async_copy(rows.at[q % _NSLOT],
                                  out_hbm.at[pl.ds(0, _MC)], sem_st).wait()


def _gather_body(a_hbm, b_hbm, ti_hbm, si_hbm, gi_hbm, gj_hbm,
                 idx, rows, sem_idx, sem_g, sem_st):
    c = lax.axis_index("c")
    s = lax.axis_index("s")

    @pl.when(c == 0)
    def _():
        _gather_pipe(a_hbm, ti_hbm, gi_hbm, idx, rows, sem_idx, sem_g,
                     sem_st, s)

    @pl.when(c == 1)
    def _():
        _gather_pipe(b_hbm, si_hbm, gj_hbm, idx, rows, sem_idx, sem_g,
                     sem_st, s)


@jax.jit
def _sc_gather(a, b, tgt, src):
    ne = tgt.shape[0]
    k = pl.kernel(
        _gather_body,
        out_type=(jax.ShapeDtypeStruct((ne, H), jnp.float32),
                  jax.ShapeDtypeStruct((ne, H), jnp.float32)),
        mesh=_mesh(),
        scratch_types=[
            pltpu.VMEM((_NSLOT, _MC), jnp.int32),
            pltpu.VMEM((_NSLOT, _MC, H), jnp.float32),
            pltpu.SemaphoreType.DMA,
            pltpu.SemaphoreType.DMA,
            pltpu.SemaphoreType.DMA,
        ],
    )
    return k(a, b, tgt, src)


def _scatter_body(ye_hbm, si_hbm, out_hbm, idx, rows, zbuf, acc, sem_ld,
                  sem_sc):
    ne = si_hbm.shape[0]
    nch = ne // _CH
    per_w = -(-nch // _NW)
    c = lax.axis_index("c")
    s = lax.axis_index("s")
    w = s * 2 + c

    @pl.loop(0, _ZR)
    def _(r):
        @pl.loop(0, H, step=16)
        def _(l):
            zbuf[pl.ds(r, 1), pl.ds(l, 16)] = jnp.zeros((1, 16), jnp.float32)

    @pl.loop(0, _RPS, step=_ZR)
    def _(r):
        pltpu.sync_copy(zbuf, acc.at[pl.ds(s * _RPS + r, _ZR)])

    @pl.when(s == 0)
    def _():
        pltpu.sync_copy(zbuf.at[pl.ds(0, _REM)], acc.at[pl.ds(16 * _RPS, _REM)])

    plsc.subcore_barrier()

    def chunk(m):
        return m * _NW + w

    def valid(m):
        return chunk(m) < nch

    def issue_load(m, k):
        base = chunk(m) * _CH
        pltpu.async_copy(si_hbm.at[pl.ds(base, _CH)], idx.at[k], sem_ld)
        pltpu.async_copy(ye_hbm.at[pl.ds(base, _CH)], rows.at[k], sem_ld)

    @pl.when(valid(0))
    def _():
        issue_load(0, 0)

    @pl.loop(0, per_w + 1)
    def _(m):
        k = m % 2
        kp = (m + 1) % 2

        @pl.when((m < per_w) & valid(m))
        def _():
            pltpu.make_async_copy(si_hbm.at[pl.ds(0, _CH)], idx.at[k],
                                  sem_ld).wait()
            pltpu.make_async_copy(ye_hbm.at[pl.ds(0, _CH)], rows.at[k],
                                  sem_ld).wait()
            pltpu.async_copy(rows.at[k], acc.at[idx.at[k]], sem_sc, add=True)

        @pl.when((m >= 1) & valid(m - 1))
        def _():
            pltpu.make_async_copy(rows.at[kp], acc.at[pl.ds(0, _CH)],
                                  sem_sc).wait()

        @pl.when((m + 1 < per_w) & valid(m + 1))
        def _():
            issue_load(m + 1, kp)

    plsc.subcore_barrier()
    pltpu.sync_copy(acc.at[pl.ds(s * _RPS, _RPS)],
                    out_hbm.at[c, pl.ds(s * _RPS, _RPS)])

    @pl.when(s == 0)
    def _():
        pltpu.sync_copy(acc.at[pl.ds(16 * _RPS, _REM)],
                        out_hbm.at[c, pl.ds(16 * _RPS, _REM)])


@jax.jit
def _sc_scatter(ye, src):
    k = pl.kernel(
        _scatter_body,
        out_type=jax.ShapeDtypeStruct((2, N, H), jnp.float32),
        mesh=_mesh(),
        scratch_types=[
            pltpu.VMEM((2, _CH), jnp.int32),
            pltpu.VMEM((2, _CH, H), jnp.float32),
            pltpu.VMEM((_ZR, H), jnp.float32),
            pltpu.VMEM_SHARED((N, H), jnp.float32),
            pltpu.SemaphoreType.DMA,
            pltpu.SemaphoreType.DMA,
        ],
    )
    return k(ye, src)


def _b(v):
    return v.reshape(1, H)


_NP = 2
_EH = E // _NP


def kernel(x, edge_attr, edge_index, params):
    f32 = jnp.float32
    ngrid = N // _BN
    eg_h = _EH // _BE
    src = [lax.slice_in_dim(edge_index[0], h * _EH, (h + 1) * _EH)
           for h in range(_NP)]
    tgt = [lax.slice_in_dim(edge_index[1], h * _EH, (h + 1) * _EH)
           for h in range(_NP)]

    def enc(inp, p, block, grid, fin, nrows, blk_off=0):
        return _tc_call(
            _enc_body, grid,
            [pl.BlockSpec((block, fin), lambda i: (i + blk_off, 0)),
             _full((fin, H)), _full((1, H)),
             _full((H, H)), _full((1, H)), _full((1, H)), _full((1, H))],
            _rows(block, H), jax.ShapeDtypeStruct((nrows, H), f32),
        )(inp, p["l1"]["w"], _b(p["l1"]["b"]), p["l2"]["w"], _b(p["l2"]["b"]),
          _b(p["g"]), _b(p["beta"]))

    xn = enc(x, params["node_enc"], _BN, ngrid, x.shape[1], N)
    ci_e = edge_attr.shape[1]
    xe = [enc(edge_attr, params["edge_enc"], _BE, eg_h, ci_e, _EH,
              blk_off=h * eg_h) for h in range(_NP)]

    for lp in params["layers"]:
        ew = lp["edge_mlp"]
        w1 = ew["l1"]["w"]
        a_tab, b_tab = _tc_call(
            _pre_body, ngrid,
            [_rows(_BN, H), _full((H, H)), _full((H, H)), _full((1, H))],
            (_rows(_BN, H), _rows(_BN, H)),
            (jax.ShapeDtypeStruct((N, H), f32),
             jax.ShapeDtypeStruct((N, H), f32)),
        )(xn, w1[:H], w1[H:2 * H], _b(ew["l1"]["b"]))

        g_h = [_sc_gather(a_tab, b_tab, tgt[h], src[h]) for h in range(_NP)]

        ye = [_tc_call(
            _edge_body, eg_h,
            [_rows(_BE, H)] * 3 + [_full((H, H)), _full((H, H)),
                                   _full((1, H)), _full((1, H)), _full((1, H))],
            _rows(_BE, H), jax.ShapeDtypeStruct((_EH, H), f32),
        )(g_h[h][0], g_h[h][1], xe[h], w1[2 * H:], ew["l2"]["w"],
          _b(ew["l2"]["b"]), _b(ew["g"]), _b(ew["beta"])) for h in range(_NP)]

        p_sum = [_sc_scatter(ye[h], src[h]) for h in range(_NP)]

        nw = lp["node_mlp"]
        wn1 = nw["l1"]["w"]
        p_spec = pl.BlockSpec((2, _BN, H), lambda i: (0, i, 0))
        xn = _tc_call(
            _node_body, ngrid,
            [p_spec] * _NP + [_rows(_BN, H),
             _full((H, H)), _full((H, H)),
             _full((1, H)), _full((H, H)), _full((1, H)), _full((1, H)),
             _full((1, H))],
            _rows(_BN, H), jax.ShapeDtypeStruct((N, H), f32),
        )(p_sum[0], p_sum[1], xn, wn1[:H], wn1[H:],
          _b(nw["l1"]["b"]), nw["l2"]["w"], _b(nw["l2"]["b"]), _b(nw["g"]),
          _b(nw["beta"]))
        xe = ye

    dec = params["dec"]
    out = _tc_call(
        _dec_body, ngrid,
        [_rows(_BN, H), _full((H, H)), _full((1, H)), _full((H, H)),
         _full((1, H))],
        _rows(_BN, H), jax.ShapeDtypeStruct((N, dec["l2"]["w"].shape[1]), f32),
    )(xn, dec["l1"]["w"], _b(dec["l1"]["b"]), dec["l2"]["w"],
      _b(dec["l2"]["b"]))
    return out
